# baseline (speedup 1.0000x reference)
import jax
import jax.numpy as jnp
from jax.experimental import pallas as pl


def _conv1d(x, W, b=None):
    out = jnp.einsum('bcn,oc->bon', x, W)
    if b is not None:
        out = out + b[None, :, None]
    return out


def _instnorm(x, eps=1e-5):
    m = jnp.mean(x, axis=2, keepdims=True)
    v = jnp.var(x, axis=2, keepdims=True)
    return (x - m) / jnp.sqrt(v + eps)


def _leaky(x):
    return jax.nn.leaky_relu(x, 0.2)


def _mesh_conv(x, nb, W, b):
    B, C, N = x.shape
    def g(k):
        idx = jnp.broadcast_to(nb[:, None, :, k], (B, C, N))
        return jnp.take_along_axis(x, idx, axis=2)
    n1, n2, n3, n4 = g(0), g(1), g(2), g(3)
    feats = jnp.stack([x, n1 + n3, jnp.abs(n1 - n3), n2 + n4, jnp.abs(n2 - n4)], axis=-1)
    return jnp.einsum('bcnk,ock->bon', feats, W) + b[None, :, None]


def _attn(x0, x1, nb, p):
    xa = _conv1d(x0, p['Wa'], p['ba'])
    xb = _conv1d(x1, p['Wb'], p['bb'])
    x_all = jnp.concatenate([xa, xb], axis=1)
    a_local = _conv1d(x_all, p['Wal'], p['bal'])
    a_tri = _mesh_conv(x_all, nb, p['Wat'], p['bat'])
    xa2 = jnp.concatenate([a_local, a_tri], axis=1)
    b_local = _conv1d(x_all, p['Wbl'], p['bbl'])
    b_tri = _mesh_conv(x_all, nb, p['Wbt'], p['bbt'])
    xb2 = jnp.concatenate([b_local, b_tri], axis=1)
    wa = jax.nn.sigmoid(_instnorm(_conv1d(xa2, p['Waf'], p['baf'])))[:, None]
    wb = jax.nn.sigmoid(_instnorm(_conv1d(xb2, p['Wbf'], p['bbf'])))[:, None]
    w = jax.nn.softmax(jnp.concatenate([wa, wb], axis=1), axis=1)
    return x0 * w[:, 0] + x1 * w[:, 1]


def _final_convs_pallas(fuse, W_ca, W_cb, W_out, b_out):
    B, C, N = fuse.shape

    def body(f_ref, wca_ref, wcb_ref, wout_ref, bout_ref, o_ref):
        f = f_ref[0]
        h = jnp.dot(wca_ref[...], f, preferred_element_type=jnp.float32)
        m = jnp.mean(h, axis=1, keepdims=True)
        v = jnp.mean((h - m) * (h - m), axis=1, keepdims=True)
        h = (h - m) * jax.lax.rsqrt(v + 1e-5)
        h = jnp.where(h >= 0, h, 0.2 * h)
        h2 = jnp.dot(wcb_ref[...], h, preferred_element_type=jnp.float32)
        m = jnp.mean(h2, axis=1, keepdims=True)
        v = jnp.mean((h2 - m) * (h2 - m), axis=1, keepdims=True)
        h2 = (h2 - m) * jax.lax.rsqrt(v + 1e-5)
        h2 = jnp.where(h2 >= 0, h2, 0.2 * h2)
        o = jnp.dot(wout_ref[...], h2, preferred_element_type=jnp.float32)
        o_ref[0] = o + bout_ref[...].reshape(-1, 1)

    out = pl.pallas_call(
        body,
        grid=(B,),
        in_specs=[
            pl.BlockSpec((1, C, N), lambda b: (b, 0, 0)),
            pl.BlockSpec((64, C), lambda b: (0, 0)),
            pl.BlockSpec((32, 64), lambda b: (0, 0)),
            pl.BlockSpec((3, 32), lambda b: (0, 0)),
            pl.BlockSpec((3,), lambda b: (0,)),
        ],
        out_specs=pl.BlockSpec((1, 3, N), lambda b: (b, 0, 0)),
        out_shape=jax.ShapeDtypeStruct((B, 3, N), jnp.float32),
    )(fuse, W_ca, W_cb, W_out, b_out)
    return out


def kernel(x, label, mesh_nb, pool_idx, unpool_idx, params):
    p = params
    B, _, N = x.shape
    NP = pool_idx.shape[1]
    nb = mesh_nb
    one_hot = jax.nn.one_hot(label + 1, 4, dtype=jnp.float32)
    new_label = jnp.transpose(one_hot, (0, 2, 1))[:, 1:, :]
    x_edge = x[:, :5, :]
    x_pos = x[:, 5:, :]
    new_x = jnp.concatenate([x_pos, new_label], axis=1)
    fe = jnp.take_along_axis(new_x, jnp.broadcast_to(pool_idx[:, None, :], (B, 6, NP)), axis=2)
    x_pool = fe[:, :-3, :]
    label_one_hot = jnp.zeros((B, 16), dtype=jnp.float32).at[:, 0].set(1.0)
    seg_pred = (jnp.einsum('bcn,oc->bon', x_pool, p['W_dg']) + p['b_dg'][None, :, None]
                + (label_one_hot @ p['W_lab'].T)[:, :, None])
    softmax_out = jax.nn.softmax(seg_pred, axis=1)
    predicted = jnp.argmax(softmax_out, axis=1)
    one_hot_enc = jnp.transpose(jax.nn.one_hot(predicted, 3, dtype=jnp.float32), (0, 2, 1))
    new_x2 = jnp.concatenate([softmax_out, one_hot_enc], axis=1)
    fe2 = jnp.take_along_axis(new_x2, jnp.broadcast_to(unpool_idx[:, None, :], (B, 6, N)), axis=2)
    fe_low = fe2[:, -3:, :]
    counts = jnp.sum((label != -1).astype(jnp.float32), axis=1)[:, None]
    counts = (counts - 3000.0) / 19000.0
    h = counts @ p['W_c1'].T + p['b_c1']
    h = h @ p['W_c2'].T + p['b_c2']
    counts_feature = jax.nn.elu(h)
    up_label = jnp.where(fe_low > 0, 1.0, fe_low)
    rounded = jnp.round(up_label)
    coarse = jnp.round(fe_low)
    feature_0 = _conv1d(x_edge, p['W_gf'], p['b_gf'])
    cf = jnp.broadcast_to(counts_feature[:, :, None], (B, 3, N))
    feat_d = jnp.concatenate([feature_0, cf, rounded], axis=1)
    feat_e = jnp.concatenate([feature_0, cf, coarse], axis=1)
    def branch(f, w1, b1, w2, b2, w3, b3):
        a = _leaky(_instnorm(_mesh_conv(f, nb, w1, b1)))
        b_ = _leaky(_instnorm(_mesh_conv(a, nb, w2, b2)))
        c = _leaky(_instnorm(_mesh_conv(b_, nb, w3, b3)))
        return jnp.concatenate([a, b_, c], axis=1)
    fea_d = branch(feat_d, p['W1d'], p['b1d'], p['W2d'], p['b2d'], p['W3d'], p['b3d'])
    fea_e = branch(feat_e, p['W1e'], p['b1e'], p['W2e'], p['b2e'], p['W3e'], p['b3e'])
    dial_out = _conv1d(fea_d, p['W_od'], p['b_od'])
    eros_out = _conv1d(fea_e, p['W_oe'], p['b_oe'])
    fuse = _attn(fea_d, fea_e, nb, p)
    fe_out = _final_convs_pallas(fuse, p['W_ca'], p['W_cb'], p['W_out'], p['b_out'])
    return (fe_out, seg_pred, seg_pred, fe_low, rounded, coarse, dial_out, eros_out)
